# Initial kernel scaffold; baseline (speedup 1.0000x reference)
import jax
import jax.numpy as jnp
from jax.experimental import pallas as pl


def kernel(x, W_router, b_router, W_noise, b_noise, W_skip, b_skip):
    raise NotImplementedError("write your pallas kernel here")


# speedup vs baseline: 3.2703x; 3.2703x over previous
import functools

import jax
import jax.numpy as jnp
from jax.experimental import pallas as pl

N_TOK = 8192
D = 4096
E = 64
TOP_K = 8
BLK = 256
NEG_INF = float("-inf")


def _router_kernel(x_ref, wcat_ref, bcat_ref, wskip_ref, bskip_ref, eps_ref,
                   router_ref, idx_ref, skip_ref):
    x = x_ref[...]
    wcat = wcat_ref[...]
    logits_all = jax.lax.dot_general(
        x, wcat, (((1,), (0,)), ((), ())),
        preferred_element_type=jnp.float32) + bcat_ref[...]
    logits = logits_all[:, :E]
    noise_logits = logits_all[:, E:]

    noise = eps_ref[...] * jax.nn.softplus(noise_logits)
    nl = logits + noise

    iota = jax.lax.broadcasted_iota(jnp.int32, (BLK, E), 1)
    vals = []
    idxs = []
    cur = nl
    for _ in range(TOP_K):
        m = jnp.max(cur, axis=1, keepdims=True)
        is_m = cur == m
        idx = jnp.min(jnp.where(is_m, iota, E), axis=1, keepdims=True)
        vals.append(m)
        idxs.append(idx)
        cur = jnp.where(iota == idx, NEG_INF, cur)

    exps = [jnp.exp(v - vals[0]) for v in vals]
    denom = functools.reduce(lambda a, b: a + b, exps)
    acc = jnp.zeros((BLK, E), jnp.float32)
    for k in range(TOP_K):
        acc = jnp.where(iota == idxs[k], exps[k] / denom, acc)
    router_ref[...] = acc
    idx_ref[...] = jnp.concatenate(idxs, axis=1)

    w = wskip_ref[...].reshape(1, D)
    s = jnp.sum(x * w, axis=1, keepdims=True) + bskip_ref[...]
    skip_ref[...] = jax.nn.sigmoid(s)


def kernel(x, W_router, b_router, W_noise, b_noise, W_skip, b_skip):
    with jax.ensure_compile_time_eval():
        eps = jax.random.normal(jax.random.key(42), (N_TOK, E), jnp.float32)

    wcat = jnp.concatenate([W_router, W_noise], axis=1)
    bcat = jnp.concatenate([b_router, b_noise])[None, :]

    grid = N_TOK // BLK
    router_out, indices, skip_prob = pl.pallas_call(
        _router_kernel,
        grid=(grid,),
        in_specs=[
            pl.BlockSpec((BLK, D), lambda i: (i, 0)),
            pl.BlockSpec((D, 2 * E), lambda i: (0, 0)),
            pl.BlockSpec((1, 2 * E), lambda i: (0, 0)),
            pl.BlockSpec((D, 1), lambda i: (0, 0)),
            pl.BlockSpec((1, 1), lambda i: (0, 0)),
            pl.BlockSpec((BLK, E), lambda i: (i, 0)),
        ],
        out_specs=[
            pl.BlockSpec((BLK, E), lambda i: (i, 0)),
            pl.BlockSpec((BLK, TOP_K), lambda i: (i, 0)),
            pl.BlockSpec((BLK, 1), lambda i: (i, 0)),
        ],
        out_shape=[
            jax.ShapeDtypeStruct((N_TOK, E), jnp.float32),
            jax.ShapeDtypeStruct((N_TOK, TOP_K), jnp.int32),
            jax.ShapeDtypeStruct((N_TOK, 1), jnp.float32),
        ],
    )(x, wcat, bcat, W_skip, b_skip[None, :], eps)
    return router_out, indices, skip_prob
